# baseline (speedup 1.0000x reference)
import functools

import jax
import jax.numpy as jnp
from jax import lax
from jax.experimental import pallas as pl
from jax.experimental.pallas import tpu as pltpu
from jax.experimental.pallas import tpu_sc as plsc

NC = 2
NS = 16
K = 96
NB = 4
BR = 400


def _round_up(a, m):
    return (a + m - 1) // m * m


def _pad_edges(src, dst, n, e, n_pad, e_pad):
    pad = e_pad - e
    dummy = n + (jnp.arange(pad, dtype=src.dtype) % (n_pad - n))
    srcp = jnp.concatenate([src, jnp.zeros((pad,), src.dtype)])
    dstp = jnp.concatenate([dst, dummy])
    return srcp, dstp


def _make_agg(n, n_pad, e_pad, d):
    nw = NC * NS
    epw = e_pad // nw
    chunks = epw // K
    groups = chunks // NB
    stripe = n_pad // NS
    mesh = plsc.VectorSubcoreMesh(core_axis_name="c", subcore_axis_name="s")

    @functools.partial(
        pl.kernel,
        out_type=jax.ShapeDtypeStruct((NC * n_pad, d), jnp.float32),
        mesh=mesh,
        scratch_types=(
            [pltpu.VMEM((K,), jnp.int32) for _ in range(2 * NB)]
            + [pltpu.VMEM((K, d), jnp.float32) for _ in range(NB)]
            + [pltpu.VMEM_SHARED((n_pad, d), jnp.float32)]
            + [pltpu.SemaphoreType.DMA for _ in range(NB)]
        ),
    )
    def k(table_hbm, src_hbm, dst_hbm, zeros_hbm, out_hbm, *scratch):
        sidx = scratch[0:NB]
        didx = scratch[NB:2 * NB]
        rows = scratch[2 * NB:3 * NB]
        acc = scratch[3 * NB]
        sems = scratch[3 * NB + 1:]
        c = lax.axis_index("c")
        s = lax.axis_index("s")
        pltpu.sync_copy(zeros_hbm, acc.at[pl.ds(s * stripe, stripe)])
        plsc.subcore_barrier()
        base = (c * NS + s) * epw

        def fetch(i, b):
            off = base + i * K
            pltpu.sync_copy(src_hbm.at[pl.ds(off, K)], sidx[b])
            pltpu.sync_copy(dst_hbm.at[pl.ds(off, K)], didx[b])
            pltpu.async_copy(table_hbm.at[sidx[b]], rows[b], sems[b])

        def drain_scatter(b):
            pltpu.make_async_copy(table_hbm.at[sidx[b]], rows[b], sems[b]).wait()
            pltpu.sync_copy(rows[b], acc.at[didx[b]], add=True)

        for b in range(NB):
            fetch(b, b)

        def body(g, carry):
            i0 = g * NB
            for b in range(NB):
                drain_scatter(b)

                @pl.when(g < groups - 1)
                def _():
                    fetch(i0 + NB + b, b)

            return carry

        lax.fori_loop(0, groups, body, 0)
        plsc.subcore_barrier()
        pltpu.sync_copy(acc.at[pl.ds(s * stripe, stripe)],
                        out_hbm.at[pl.ds(c * n_pad + s * stripe, stripe)])

    return k


def _make_deg(n_pad, e_pad, d):
    nw = NC * NS
    epw = e_pad // nw
    cpw = epw // K
    stripe = n_pad // NS
    mesh = plsc.VectorSubcoreMesh(core_axis_name="c", subcore_axis_name="s")

    @functools.partial(
        pl.kernel,
        out_type=jax.ShapeDtypeStruct((NC * n_pad, d), jnp.float32),
        mesh=mesh,
        scratch_types=[
            pltpu.VMEM((K,), jnp.int32), pltpu.VMEM((K,), jnp.int32),
            pltpu.VMEM((K, d), jnp.float32),
            pltpu.VMEM_SHARED((n_pad, d), jnp.float32),
        ],
    )
    def k(dst_hbm, ones_hbm, zeros_hbm, out_hbm, didx0, didx1, ones_v, acc):
        c = lax.axis_index("c")
        s = lax.axis_index("s")
        pltpu.sync_copy(ones_hbm, ones_v)
        pltpu.sync_copy(zeros_hbm, acc.at[pl.ds(s * stripe, stripe)])
        plsc.subcore_barrier()
        base = (c * NS + s) * epw

        def body(j, carry):
            off = base + 2 * j * K
            pltpu.sync_copy(dst_hbm.at[pl.ds(off, K)], didx0)
            pltpu.sync_copy(dst_hbm.at[pl.ds(off + K, K)], didx1)
            pltpu.sync_copy(ones_v, acc.at[didx0], add=True)
            pltpu.sync_copy(ones_v, acc.at[didx1], add=True)
            return carry

        lax.fori_loop(0, cpw // 2, body, 0)
        plsc.subcore_barrier()
        pltpu.sync_copy(acc.at[pl.ds(s * stripe, stripe)],
                        out_hbm.at[pl.ds(c * n_pad + s * stripe, stripe)])

    return k


def _dinv_of(degp0, degp1):
    deg = degp0[:, 0:1] + degp1[:, 0:1] + 1.0
    return 1.0 / jnp.sqrt(jnp.maximum(deg, 1.0))


def _prep_body(degp_ref, x_ref, xs_ref):
    dinv = _dinv_of(degp_ref[0], degp_ref[1])
    xs_ref[...] = x_ref[...] * dinv


def _l1_body(degp_ref, rawp_ref, xs_ref, w_ref, b_ref, hs_ref):
    dinv = _dinv_of(degp_ref[0], degp_ref[1])
    agg = (rawp_ref[0] + rawp_ref[1] + xs_ref[...]) * dinv
    h = jnp.dot(agg, w_ref[...], preferred_element_type=jnp.float32) + b_ref[...]
    hs_ref[...] = jnp.maximum(h, 0.0) * dinv


def _l2_body(degp_ref, rawp_ref, hs_ref, wmu_ref, bmu_ref, wls_ref, bls_ref,
             mu_ref, ls_ref):
    dinv = _dinv_of(degp_ref[0], degp_ref[1])
    agg = (rawp_ref[0] + rawp_ref[1] + hs_ref[...]) * dinv
    mu_ref[...] = jnp.dot(agg, wmu_ref[...], preferred_element_type=jnp.float32) + bmu_ref[...]
    ls_ref[...] = jnp.dot(agg, wls_ref[...], preferred_element_type=jnp.float32) + bls_ref[...]


def _row_spec(d):
    return pl.BlockSpec((BR, d), lambda i: (i, 0))


def _part_spec(d):
    return pl.BlockSpec((NC, BR, d), lambda i: (0, i, 0))


def _full_spec(shape):
    return pl.BlockSpec(shape, lambda i: tuple(0 for _ in shape))


def kernel(x, edge_index, W1, b1, W_mu, b_mu, W_ls, b_ls):
    n, d_in = x.shape
    e = edge_index.shape[1]
    d_hid = W1.shape[1]
    d_out = W_mu.shape[1]
    nw = NC * NS

    n_pad = _round_up(n + 1, 128)
    e_pad = _round_up(e, nw * K * NB)
    stripe = n_pad // NS
    grid = (n // BR,)

    srcp, dstp = _pad_edges(edge_index[0], edge_index[1], n, e, n_pad, e_pad)
    zeros_d = jnp.zeros((stripe, d_in), jnp.float32)
    ones_k = jnp.ones((K, d_in), jnp.float32)

    agg_fn = _make_agg(n, n_pad, e_pad, d_in)
    degp = _make_deg(n_pad, e_pad, d_in)(dstp, ones_k, zeros_d).reshape(NC, n_pad, d_in)

    xs = pl.pallas_call(
        _prep_body, grid=grid,
        in_specs=[_part_spec(d_in), _row_spec(d_in)],
        out_specs=_row_spec(d_in),
        out_shape=jax.ShapeDtypeStruct((n, d_in), jnp.float32),
    )(degp, x)

    raw1 = agg_fn(xs, srcp, dstp, zeros_d).reshape(NC, n_pad, d_in)

    hs = pl.pallas_call(
        _l1_body, grid=grid,
        in_specs=[_part_spec(d_in), _part_spec(d_in), _row_spec(d_in),
                  _full_spec((d_in, d_hid)), _full_spec((1, d_hid))],
        out_specs=_row_spec(d_hid),
        out_shape=jax.ShapeDtypeStruct((n, d_hid), jnp.float32),
    )(degp, raw1, xs, W1, b1.reshape(1, d_hid))

    raw2 = agg_fn(hs, srcp, dstp, zeros_d).reshape(NC, n_pad, d_hid)

    mu, ls = pl.pallas_call(
        _l2_body, grid=grid,
        in_specs=[_part_spec(d_in), _part_spec(d_hid), _row_spec(d_hid),
                  _full_spec((d_hid, d_out)), _full_spec((1, d_out)),
                  _full_spec((d_hid, d_out)), _full_spec((1, d_out))],
        out_specs=[_row_spec(d_out), _row_spec(d_out)],
        out_shape=[jax.ShapeDtypeStruct((n, d_out), jnp.float32),
                   jax.ShapeDtypeStruct((n, d_out), jnp.float32)],
    )(degp, raw2, hs, W_mu, b_mu.reshape(1, d_out), W_ls, b_ls.reshape(1, d_out))

    return (mu, ls)


# speedup vs baseline: 2.3382x; 2.3382x over previous
import functools

import jax
import jax.numpy as jnp
from jax import lax
from jax.experimental import pallas as pl
from jax.experimental.pallas import tpu as pltpu
from jax.experimental.pallas import tpu_sc as plsc

NC = 2
NS = 16
K = 120
NB = 3
BR = 400


def _round_up(a, m):
    return (a + m - 1) // m * m


def _pad_edges(src, dst, n, e, n_pad, e_pad):
    pad = e_pad - e
    dummy = n + (jnp.arange(pad, dtype=src.dtype) % (n_pad - n))
    srcp = jnp.concatenate([src, jnp.zeros((pad,), src.dtype)])
    dstp = jnp.concatenate([dst, dummy])
    return srcp, dstp


def _make_agg(n, n_pad, e_pad, d):
    nw = NC * NS
    epw = e_pad // nw
    chunks = epw // K
    groups = chunks // NB
    stripe = n_pad // NS
    mesh = plsc.VectorSubcoreMesh(core_axis_name="c", subcore_axis_name="s")

    @functools.partial(
        pl.kernel,
        out_type=jax.ShapeDtypeStruct((NC * n_pad, d), jnp.float32),
        mesh=mesh,
        scratch_types=(
            [pltpu.VMEM((K,), jnp.int32) for _ in range(2 * NB)]
            + [pltpu.VMEM((K, d), jnp.float32) for _ in range(NB)]
            + [pltpu.VMEM_SHARED((n_pad, d), jnp.float32)]
            + [pltpu.SemaphoreType.DMA for _ in range(NB)]
        ),
    )
    def k(table_hbm, src_hbm, dst_hbm, zeros_hbm, out_hbm, *scratch):
        sidx = scratch[0:NB]
        didx = scratch[NB:2 * NB]
        rows = scratch[2 * NB:3 * NB]
        acc = scratch[3 * NB]
        sems = scratch[3 * NB + 1:]
        c = lax.axis_index("c")
        s = lax.axis_index("s")
        pltpu.sync_copy(zeros_hbm, acc.at[pl.ds(s * stripe, stripe)])
        plsc.subcore_barrier()
        base = (c * NS + s) * epw

        def fetch(i, b):
            off = base + i * K
            pltpu.sync_copy(src_hbm.at[pl.ds(off, K)], sidx[b])
            pltpu.sync_copy(dst_hbm.at[pl.ds(off, K)], didx[b])
            pltpu.async_copy(table_hbm.at[sidx[b]], rows[b], sems[b])

        def drain_scatter(b):
            pltpu.make_async_copy(table_hbm.at[sidx[b]], rows[b], sems[b]).wait()
            pltpu.sync_copy(rows[b], acc.at[didx[b]], add=True)

        for b in range(NB):
            fetch(b, b)

        def body(g, carry):
            i0 = g * NB
            for b in range(NB):
                drain_scatter(b)

                @pl.when(g < groups - 1)
                def _():
                    fetch(i0 + NB + b, b)

            return carry

        lax.fori_loop(0, groups, body, 0)
        plsc.subcore_barrier()
        pltpu.sync_copy(acc.at[pl.ds(s * stripe, stripe)],
                        out_hbm.at[pl.ds(c * n_pad + s * stripe, stripe)])

    return k


def _make_deg(n_pad, e_pad, d):
    nw = NC * NS
    epw = e_pad // nw
    cpw = epw // K
    stripe = n_pad // NS
    mesh = plsc.VectorSubcoreMesh(core_axis_name="c", subcore_axis_name="s")

    @functools.partial(
        pl.kernel,
        out_type=jax.ShapeDtypeStruct((NC * n_pad, d), jnp.float32),
        mesh=mesh,
        scratch_types=[
            pltpu.VMEM((K,), jnp.int32), pltpu.VMEM((K,), jnp.int32),
            pltpu.VMEM((K, d), jnp.float32),
            pltpu.VMEM_SHARED((n_pad, d), jnp.float32),
        ],
    )
    def k(dst_hbm, ones_hbm, zeros_hbm, out_hbm, didx0, didx1, ones_v, acc):
        c = lax.axis_index("c")
        s = lax.axis_index("s")
        pltpu.sync_copy(ones_hbm, ones_v)
        pltpu.sync_copy(zeros_hbm, acc.at[pl.ds(s * stripe, stripe)])
        plsc.subcore_barrier()
        base = (c * NS + s) * epw

        def body(j, carry):
            off = base + 2 * j * K
            pltpu.sync_copy(dst_hbm.at[pl.ds(off, K)], didx0)
            pltpu.sync_copy(dst_hbm.at[pl.ds(off + K, K)], didx1)
            pltpu.sync_copy(ones_v, acc.at[didx0], add=True)
            pltpu.sync_copy(ones_v, acc.at[didx1], add=True)
            return carry

        lax.fori_loop(0, cpw // 2, body, 0)
        plsc.subcore_barrier()
        pltpu.sync_copy(acc.at[pl.ds(s * stripe, stripe)],
                        out_hbm.at[pl.ds(c * n_pad + s * stripe, stripe)])

    return k


def _dinv_of(degp0, degp1):
    deg = degp0[:, 0:1] + degp1[:, 0:1] + 1.0
    return 1.0 / jnp.sqrt(jnp.maximum(deg, 1.0))


def _mm_body(x_ref, w_ref, y_ref):
    y_ref[...] = jnp.dot(x_ref[...], w_ref[...],
                         preferred_element_type=jnp.float32)


def _prep_body(degp_ref, x_ref, xs_ref):
    dinv = _dinv_of(degp_ref[0], degp_ref[1])
    xs_ref[...] = x_ref[...] * dinv


def _l1_body(degp_ref, rawp_ref, ys_ref, b_ref, hs_ref):
    dinv = _dinv_of(degp_ref[0], degp_ref[1])
    h = (rawp_ref[0] + rawp_ref[1] + ys_ref[...]) * dinv + b_ref[...]
    hs_ref[...] = jnp.maximum(h, 0.0) * dinv


def _l2_body(degp_ref, rawp_ref, hs_ref, wmu_ref, bmu_ref, wls_ref, bls_ref,
             mu_ref, ls_ref):
    dinv = _dinv_of(degp_ref[0], degp_ref[1])
    agg = (rawp_ref[0] + rawp_ref[1] + hs_ref[...]) * dinv
    mu_ref[...] = jnp.dot(agg, wmu_ref[...], preferred_element_type=jnp.float32) + bmu_ref[...]
    ls_ref[...] = jnp.dot(agg, wls_ref[...], preferred_element_type=jnp.float32) + bls_ref[...]


def _row_spec(d):
    return pl.BlockSpec((BR, d), lambda i: (i, 0))


def _part_spec(d):
    return pl.BlockSpec((NC, BR, d), lambda i: (0, i, 0))


def _full_spec(shape):
    return pl.BlockSpec(shape, lambda i: tuple(0 for _ in shape))


def kernel(x, edge_index, W1, b1, W_mu, b_mu, W_ls, b_ls):
    n, d_in = x.shape
    e = edge_index.shape[1]
    d_hid = W1.shape[1]
    d_out = W_mu.shape[1]
    nw = NC * NS

    n_pad = _round_up(n + 1, 128)
    e_pad = _round_up(e, nw * K * NB)
    stripe = n_pad // NS
    grid = (n // BR,)

    srcp, dstp = _pad_edges(edge_index[0], edge_index[1], n, e, n_pad, e_pad)
    zeros_d = jnp.zeros((stripe, d_in), jnp.float32)
    zeros_h = jnp.zeros((stripe, d_hid), jnp.float32)
    ones_k = jnp.ones((K, d_in), jnp.float32)

    y = pl.pallas_call(
        _mm_body, grid=grid,
        in_specs=[_row_spec(d_in), _full_spec((d_in, d_hid))],
        out_specs=_row_spec(d_hid),
        out_shape=jax.ShapeDtypeStruct((n, d_hid), jnp.float32),
    )(x, W1)

    agg_fn = _make_agg(n, n_pad, e_pad, d_hid)
    degp = _make_deg(n_pad, e_pad, d_in)(dstp, ones_k, zeros_d).reshape(NC, n_pad, d_in)

    ys = pl.pallas_call(
        _prep_body, grid=grid,
        in_specs=[_part_spec(d_in), _row_spec(d_hid)],
        out_specs=_row_spec(d_hid),
        out_shape=jax.ShapeDtypeStruct((n, d_hid), jnp.float32),
    )(degp, y)

    raw1 = agg_fn(ys, srcp, dstp, zeros_h).reshape(NC, n_pad, d_hid)

    hs = pl.pallas_call(
        _l1_body, grid=grid,
        in_specs=[_part_spec(d_in), _part_spec(d_hid), _row_spec(d_hid),
                  _full_spec((1, d_hid))],
        out_specs=_row_spec(d_hid),
        out_shape=jax.ShapeDtypeStruct((n, d_hid), jnp.float32),
    )(degp, raw1, ys, b1.reshape(1, d_hid))

    raw2 = agg_fn(hs, srcp, dstp, zeros_h).reshape(NC, n_pad, d_hid)

    mu, ls = pl.pallas_call(
        _l2_body, grid=grid,
        in_specs=[_part_spec(d_in), _part_spec(d_hid), _row_spec(d_hid),
                  _full_spec((d_hid, d_out)), _full_spec((1, d_out)),
                  _full_spec((d_hid, d_out)), _full_spec((1, d_out))],
        out_specs=[_row_spec(d_out), _row_spec(d_out)],
        out_shape=[jax.ShapeDtypeStruct((n, d_out), jnp.float32),
                   jax.ShapeDtypeStruct((n, d_out), jnp.float32)],
    )(degp, raw2, hs, W_mu, b_mu.reshape(1, d_out), W_ls, b_ls.reshape(1, d_out))

    return (mu, ls)
